# baseline (speedup 1.0000x reference)
import jax
import jax.numpy as jnp
import numpy as np
from jax.experimental import pallas as pl
from jax.experimental.pallas import tpu as pltpu

D = 2048
H3 = 192
BR = 1024
NCHUNK = 8
CK = D // NCHUNK


def _fused_kernel(*refs):
    x_refs = refs[:NCHUNK]
    (tail_ref, w0_ref, b0_ref, w1_ref, b1_ref, w2_ref,
     b2_ref, seg_ref, pi_ref, v_ref) = refs[NCHUNK:]
    gate = tail_ref[:, :3]
    acc = b0_ref[:, :]
    for k in range(NCHUNK):
        xk = x_refs[k][:, :].astype(jnp.bfloat16)
        wk = w0_ref[k * CK:(k + 1) * CK, :]
        acc = acc + jnp.dot(xk, wk, preferred_element_type=jnp.float32)
    h = jnp.tanh(acc)
    h = jnp.tanh(jnp.dot(h, w1_ref[:, :], preferred_element_type=jnp.float32)
                 + b1_ref[:, :])
    o = jnp.dot(h, w2_ref[:, :], preferred_element_type=jnp.float32) + b2_ref[:, :]
    e = jnp.exp(o[:, 0:64])
    s = jnp.dot(e, seg_ref[:, :], preferred_element_type=jnp.float32)
    p = e / s
    mask = jnp.all(jnp.abs(gate) <= 0.1, axis=-1, keepdims=True)
    col = jax.lax.broadcasted_iota(jnp.int32, (1, 64), 1) < 32
    pi_ref[:, :] = jnp.where(mask == col, p, 0.0)
    v_ref[:, :] = o[:, 64:65]


def kernel(observation, prev_action, prev_reward,
           pf_W0, pf_b0, pf_W1, pf_b1, pf_W2, pf_b2,
           pc_W0, pc_b0, pc_W1, pc_b1, pc_W2, pc_b2,
           v_W0, v_b0, v_W1, v_b1, v_W2, v_b2):
    B = observation.shape[0]
    f32 = jnp.float32
    observation = jnp.pad(observation, ((0, 0), (0, 125)))

    W0 = jnp.concatenate([pf_W0, pc_W0, v_W0], axis=1).astype(jnp.bfloat16)
    b0 = jnp.concatenate([pf_b0, pc_b0, v_b0])[None, :]
    W1 = jax.scipy.linalg.block_diag(pf_W1, pc_W1, v_W1)
    b1 = jnp.concatenate([pf_b1, pc_b1, v_b1])[None, :]
    W2 = jnp.zeros((H3, 128), dtype=f32)
    W2 = W2.at[0:64, 0:32].set(pf_W2)
    W2 = W2.at[64:128, 32:64].set(pc_W2)
    W2 = W2.at[128:192, 64:65].set(v_W2)
    b2 = jnp.zeros((128,), dtype=f32)
    b2 = b2.at[0:32].set(pf_b2)
    b2 = b2.at[32:64].set(pc_b2)
    b2 = b2.at[64].set(v_b2[0])
    b2 = b2[None, :]
    seg = jax.scipy.linalg.block_diag(jnp.ones((32, 32), f32),
                                      jnp.ones((32, 32), f32))

    grid = (B // BR,)
    rep = lambda i: (0, 0)
    pi, v = pl.pallas_call(
        _fused_kernel,
        grid=grid,
        in_specs=[
            *[pl.BlockSpec((BR, CK), lambda i, k=k: (i, k))
              for k in range(NCHUNK)],
            pl.BlockSpec((BR, 128), lambda i: (i, D // 128)),
            pl.BlockSpec((D, H3), rep),
            pl.BlockSpec((1, H3), rep),
            pl.BlockSpec((H3, H3), rep),
            pl.BlockSpec((1, H3), rep),
            pl.BlockSpec((H3, 128), rep),
            pl.BlockSpec((1, 128), rep),
            pl.BlockSpec((64, 64), rep),
        ],
        out_specs=[
            pl.BlockSpec((BR, 64), lambda i: (i, 0)),
            pl.BlockSpec((BR, 1), lambda i: (i, 0)),
        ],
        out_shape=[
            jax.ShapeDtypeStruct((B, 64), f32),
            jax.ShapeDtypeStruct((B, 1), f32),
        ],
        compiler_params=pltpu.CompilerParams(
            dimension_semantics=("parallel",)),
    )(*([observation] * NCHUNK), observation, W0, b0, W1, b1, W2, b2, seg)
    return (pi, v[:, 0])


# speedup vs baseline: 1.0681x; 1.0681x over previous
import jax
import jax.numpy as jnp
import numpy as np
from jax.experimental import pallas as pl
from jax.experimental.pallas import tpu as pltpu

D = 2048
BR = 1024


def _fused_kernel(x_ref, tail_ref,
                  w0f_ref, w0c_ref, w0v_ref,
                  w1f_ref, w1c_ref, w1v_ref,
                  w2f_ref, w2c_ref, w2v_ref,
                  b0f_ref, b0c_ref, b0v_ref,
                  b1f_ref, b1c_ref, b1v_ref,
                  b2f_ref, b2c_ref, b2v_ref,
                  pi_ref, v_ref):
    f32 = jnp.float32
    xb = x_ref[:, :].astype(jnp.bfloat16)
    gate = tail_ref[:, :3]

    def mlp2(w0_ref, b0_ref, w1_ref, b1_ref):
        w0 = w0_ref[:, :].astype(jnp.bfloat16)
        h = jnp.tanh(jnp.dot(xb, w0, preferred_element_type=f32)
                     + b0_ref[:, :])
        return jnp.tanh(jnp.dot(h, w1_ref[:, :], preferred_element_type=f32)
                        + b1_ref[:, :])

    hf = mlp2(w0f_ref, b0f_ref, w1f_ref, b1f_ref)
    hc = mlp2(w0c_ref, b0c_ref, w1c_ref, b1c_ref)
    hv = mlp2(w0v_ref, b0v_ref, w1v_ref, b1v_ref)

    of = jnp.dot(hf, w2f_ref[:, :], preferred_element_type=f32) + b2f_ref[:, :]
    oc = jnp.dot(hc, w2c_ref[:, :], preferred_element_type=f32) + b2c_ref[:, :]
    ov = jnp.dot(hv, w2v_ref[:, :], preferred_element_type=f32) + b2v_ref[:, :]

    ones32 = jnp.ones((32, 32), f32)
    ef = jnp.exp(of)
    ec = jnp.exp(oc)
    sf = jnp.dot(ef, ones32, preferred_element_type=f32)
    sc = jnp.dot(ec, ones32, preferred_element_type=f32)
    mask = jnp.all(jnp.abs(gate) <= 0.1, axis=-1, keepdims=True)
    pi_ref[:, 0:32] = jnp.where(mask, ef / sf, 0.0)
    pi_ref[:, 32:64] = jnp.where(mask, 0.0, ec / sc)
    v_ref[:, :] = ov


def kernel(observation, prev_action, prev_reward,
           pf_W0, pf_b0, pf_W1, pf_b1, pf_W2, pf_b2,
           pc_W0, pc_b0, pc_W1, pc_b1, pc_W2, pc_b2,
           v_W0, v_b0, v_W1, v_b1, v_W2, v_b2):
    B = observation.shape[0]
    f32 = jnp.float32

    grid = (B // BR,)
    rep = lambda i: (0, 0)

    def wspec(arr):
        return pl.BlockSpec(arr.shape, rep)

    biases = [pf_b0[None, :], pc_b0[None, :], v_b0[None, :],
              pf_b1[None, :], pc_b1[None, :], v_b1[None, :],
              pf_b2[None, :], pc_b2[None, :], v_b2[None, :]]
    weights = [pf_W0, pc_W0, v_W0, pf_W1, pc_W1, v_W1, pf_W2, pc_W2, v_W2]

    pi, v = pl.pallas_call(
        _fused_kernel,
        grid=grid,
        in_specs=[
            pl.BlockSpec((BR, D), lambda i: (i, 0)),
            pl.BlockSpec((BR, 128), lambda i: (i, D // 128)),
            *[wspec(w) for w in weights],
            *[wspec(b) for b in biases],
        ],
        out_specs=[
            pl.BlockSpec((BR, 64), lambda i: (i, 0)),
            pl.BlockSpec((BR, 1), lambda i: (i, 0)),
        ],
        out_shape=[
            jax.ShapeDtypeStruct((B, 64), f32),
            jax.ShapeDtypeStruct((B, 1), f32),
        ],
        compiler_params=pltpu.CompilerParams(
            dimension_semantics=("parallel",)),
    )(observation, observation, *weights, *biases)
    return (pi, v[:, 0])


# speedup vs baseline: 2.4779x; 2.3198x over previous
import jax
import jax.numpy as jnp
import numpy as np
from jax.experimental import pallas as pl
from jax.experimental.pallas import tpu as pltpu

D = 2048
BC = 1024


def _fused_kernel(x_ref, tail_ref,
                  w0f_ref, w0c_ref, w0v_ref,
                  w1f_ref, w1c_ref, w1v_ref,
                  w2f_ref, w2c_ref, w2v_ref,
                  b0f_ref, b0c_ref, b0v_ref,
                  b1f_ref, b1c_ref, b1v_ref,
                  b2f_ref, b2c_ref,
                  pi_ref, v_ref):
    f32 = jnp.float32
    xb = x_ref[:, :].astype(jnp.bfloat16)
    gate = tail_ref[0:3, :]

    def mlp2(w0_ref, b0_ref, w1_ref, b1_ref):
        w0 = w0_ref[:, :].astype(jnp.bfloat16)
        h = jnp.tanh(jnp.dot(w0, xb, preferred_element_type=f32)
                     + b0_ref[:, :])
        return jnp.tanh(jnp.dot(w1_ref[:, :], h, preferred_element_type=f32)
                        + b1_ref[:, :])

    hf = mlp2(w0f_ref, b0f_ref, w1f_ref, b1f_ref)
    hc = mlp2(w0c_ref, b0c_ref, w1c_ref, b1c_ref)
    hv = mlp2(w0v_ref, b0v_ref, w1v_ref, b1v_ref)

    of = jnp.dot(w2f_ref[:, :], hf, preferred_element_type=f32) + b2f_ref[:, :]
    oc = jnp.dot(w2c_ref[:, :], hc, preferred_element_type=f32) + b2c_ref[:, :]
    ov = jnp.dot(w2v_ref[:, :], hv, preferred_element_type=f32)

    ones32 = jnp.ones((32, 32), f32)
    ef = jnp.exp(of)
    ec = jnp.exp(oc)
    sf = jnp.dot(ones32, ef, preferred_element_type=f32)
    sc = jnp.dot(ones32, ec, preferred_element_type=f32)
    mask = jnp.all(jnp.abs(gate) <= 0.1, axis=0, keepdims=True)
    pi_ref[0:32, :] = jnp.where(mask, ef / sf, 0.0)
    pi_ref[32:64, :] = jnp.where(mask, 0.0, ec / sc)
    v_ref[:, :] = ov


def kernel(observation, prev_action, prev_reward,
           pf_W0, pf_b0, pf_W1, pf_b1, pf_W2, pf_b2,
           pc_W0, pc_b0, pc_W1, pc_b1, pc_W2, pc_b2,
           v_W0, v_b0, v_W1, v_b1, v_W2, v_b2):
    B = observation.shape[0]
    f32 = jnp.float32

    obs_t = observation.T
    weights = [pf_W0.T, pc_W0.T, v_W0.T,
               pf_W1.T, pc_W1.T, v_W1.T,
               pf_W2.T, pc_W2.T, v_W2.T]
    biases = [pf_b0[:, None], pc_b0[:, None], v_b0[:, None],
              pf_b1[:, None], pc_b1[:, None], v_b1[:, None],
              pf_b2[:, None], pc_b2[:, None]]

    grid = (B // BC,)
    rep = lambda i: (0, 0)

    def wspec(arr):
        return pl.BlockSpec(arr.shape, rep)

    pi_t, v_t = pl.pallas_call(
        _fused_kernel,
        grid=grid,
        in_specs=[
            pl.BlockSpec((D, BC), lambda i: (0, i)),
            pl.BlockSpec((8, BC), lambda i: (D // 8, i)),
            *[wspec(w) for w in weights],
            *[wspec(b) for b in biases],
        ],
        out_specs=[
            pl.BlockSpec((64, BC), lambda i: (0, i)),
            pl.BlockSpec((1, BC), lambda i: (0, i)),
        ],
        out_shape=[
            jax.ShapeDtypeStruct((64, B), f32),
            jax.ShapeDtypeStruct((1, B), f32),
        ],
        compiler_params=pltpu.CompilerParams(
            dimension_semantics=("parallel",)),
    )(obs_t, obs_t, *weights, *biases)
    return (pi_t.T, v_t[0] + v_b2[0])


# speedup vs baseline: 3.0516x; 1.2316x over previous
import jax
import jax.numpy as jnp
import numpy as np
from jax.experimental import pallas as pl
from jax.experimental.pallas import tpu as pltpu

D = 2048
BC = 1024


def _fused_kernel(x_ref, tail_ref,
                  w0f_ref, w0c_ref, w0v_ref,
                  w1f_ref, w1c_ref, w1v_ref,
                  w2f_ref, w2c_ref, w2v_ref,
                  bias_ref, pi_ref, v_ref):
    f32 = jnp.float32
    xb = x_ref[:, :].astype(jnp.bfloat16)
    gate = tail_ref[0:3, :]

    def mlp2(w0_ref, b0, w1_ref, b1):
        w0 = w0_ref[:, :].astype(jnp.bfloat16)
        h = jnp.tanh(jnp.dot(w0, xb, preferred_element_type=f32) + b0)
        return jnp.tanh(jnp.dot(w1_ref[:, :], h, preferred_element_type=f32)
                        + b1)

    b = bias_ref[:, :]
    hf = mlp2(w0f_ref, b[0:64], w1f_ref, b[192:256])
    hc = mlp2(w0c_ref, b[64:128], w1c_ref, b[256:320])
    hv = mlp2(w0v_ref, b[128:192], w1v_ref, b[320:384])

    of = jnp.dot(w2f_ref[:, :], hf, preferred_element_type=f32) + b[384:416]
    oc = jnp.dot(w2c_ref[:, :], hc, preferred_element_type=f32) + b[416:448]
    ov = jnp.dot(w2v_ref[:, :], hv, preferred_element_type=f32) + b[448:449]

    ones32 = jnp.ones((32, 32), f32)
    ef = jnp.exp(of)
    ec = jnp.exp(oc)
    sf = jnp.dot(ones32, ef, preferred_element_type=f32)
    sc = jnp.dot(ones32, ec, preferred_element_type=f32)
    mask = jnp.all(jnp.abs(gate) <= 0.1, axis=0, keepdims=True)
    pf = jnp.where(mask, ef / sf, 0.0)
    pc = jnp.where(mask, 0.0, ec / sc)
    pi_t = jnp.concatenate([pf, pc], axis=0)
    pi_ref[:, :] = pi_t.T
    v_ref[:, :] = ov


def kernel(observation, prev_action, prev_reward,
           pf_W0, pf_b0, pf_W1, pf_b1, pf_W2, pf_b2,
           pc_W0, pc_b0, pc_W1, pc_b1, pc_W2, pc_b2,
           v_W0, v_b0, v_W1, v_b1, v_W2, v_b2):
    B = observation.shape[0]
    f32 = jnp.float32

    obs_t = observation.T
    weights = [pf_W0.T, pc_W0.T, v_W0.T,
               pf_W1.T, pc_W1.T, v_W1.T,
               pf_W2.T, pc_W2.T, v_W2.T]
    bias = jnp.concatenate(
        [pf_b0, pc_b0, v_b0, pf_b1, pc_b1, v_b1,
         pf_b2, pc_b2, v_b2])[:, None]

    grid = (B // BC,)
    rep = lambda i: (0, 0)

    def wspec(arr):
        return pl.BlockSpec(arr.shape, rep)

    pi, v_t = pl.pallas_call(
        _fused_kernel,
        grid=grid,
        in_specs=[
            pl.BlockSpec((D, BC), lambda i: (0, i)),
            pl.BlockSpec((8, BC), lambda i: (D // 8, i)),
            *[wspec(w) for w in weights],
            pl.BlockSpec((449, 1), rep),
        ],
        out_specs=[
            pl.BlockSpec((BC, 64), lambda i: (i, 0)),
            pl.BlockSpec((1, BC), lambda i: (0, i)),
        ],
        out_shape=[
            jax.ShapeDtypeStruct((B, 64), f32),
            jax.ShapeDtypeStruct((1, B), f32),
        ],
        compiler_params=pltpu.CompilerParams(
            dimension_semantics=("parallel",)),
    )(obs_t, obs_t, *weights, bias)
    return (pi, v_t[0])


# speedup vs baseline: 4.1740x; 1.3678x over previous
import jax
import jax.numpy as jnp
import numpy as np
from jax.experimental import pallas as pl
from jax.experimental.pallas import tpu as pltpu

D = 2048
BC = 1024


def _fused_kernel(x_ref, tail_ref,
                  w0f_ref, w0c_ref, w0v_ref,
                  w1f_ref, w1c_ref, w1v_ref,
                  w2f_ref, w2c_ref, w2v_ref,
                  bias_ref, pi_ref, v_ref):
    f32 = jnp.float32
    xb = x_ref[:, :].astype(jnp.bfloat16)
    gate = tail_ref[0:3, :]

    def mlp2(w0_ref, b0, w1_ref, b1):
        w0 = w0_ref[:, :].astype(jnp.bfloat16)
        h = jnp.tanh(jnp.dot(w0, xb, preferred_element_type=f32) + b0)
        h2 = jax.lax.dot_general(w1_ref[:, :], h, (((0,), (0,)), ((), ())),
                                 preferred_element_type=f32)
        return jnp.tanh(h2 + b1)

    b = bias_ref[:, :]
    hf = mlp2(w0f_ref, b[0:64], w1f_ref, b[192:256])
    hc = mlp2(w0c_ref, b[64:128], w1c_ref, b[256:320])
    hv = mlp2(w0v_ref, b[128:192], w1v_ref, b[320:384])

    of = jnp.dot(w2f_ref[:, :], hf, preferred_element_type=f32) + b[384:416]
    oc = jnp.dot(w2c_ref[:, :], hc, preferred_element_type=f32) + b[416:448]
    ov = jnp.dot(w2v_ref[:, :], hv, preferred_element_type=f32) + b[448:449]

    ones32 = jnp.ones((32, 32), f32)
    ef = jnp.exp(of)
    ec = jnp.exp(oc)
    sf = jnp.dot(ones32, ef, preferred_element_type=f32)
    sc = jnp.dot(ones32, ec, preferred_element_type=f32)
    mask = jnp.all(jnp.abs(gate) <= 0.1, axis=0, keepdims=True)
    pi_ref[0:32, :] = jnp.where(mask, ef / sf, 0.0)
    pi_ref[32:64, :] = jnp.where(mask, 0.0, ec / sc)
    v_ref[:, :] = ov


def kernel(observation, prev_action, prev_reward,
           pf_W0, pf_b0, pf_W1, pf_b1, pf_W2, pf_b2,
           pc_W0, pc_b0, pc_W1, pc_b1, pc_W2, pc_b2,
           v_W0, v_b0, v_W1, v_b1, v_W2, v_b2):
    B = observation.shape[0]
    f32 = jnp.float32

    obs_t = observation.T
    weights = [pf_W0.T, pc_W0.T, v_W0.T,
               pf_W1, pc_W1, v_W1,
               pf_W2.T, pc_W2.T, v_W2.T]
    bias = jnp.concatenate(
        [pf_b0, pc_b0, v_b0, pf_b1, pc_b1, v_b1,
         pf_b2, pc_b2, v_b2])[:, None]

    grid = (B // BC,)
    rep = lambda i: (0, 0)

    def wspec(arr):
        return pl.BlockSpec(arr.shape, rep)

    pi, v_t = pl.pallas_call(
        _fused_kernel,
        grid=grid,
        in_specs=[
            pl.BlockSpec((D, BC), lambda i: (0, i)),
            pl.BlockSpec((8, BC), lambda i: (D // 8, i)),
            *[wspec(w) for w in weights],
            pl.BlockSpec((449, 1), rep),
        ],
        out_specs=[
            pl.BlockSpec((64, BC), lambda i: (0, i)),
            pl.BlockSpec((1, BC), lambda i: (0, i)),
        ],
        out_shape=[
            jax.ShapeDtypeStruct((64, B), f32),
            jax.ShapeDtypeStruct((1, B), f32),
        ],
        compiler_params=pltpu.CompilerParams(
            dimension_semantics=("parallel",)),
    )(obs_t, obs_t, *weights, bias)
    return (pi.T, v_t[0])
